# Initial kernel scaffold; baseline (speedup 1.0000x reference)
import jax
import jax.numpy as jnp
from jax.experimental import pallas as pl


def kernel(x, edge_index, manual_features, W1_self, W1_neigh, b1, W2_self, W2_neigh, b2, W3, b3, W4, b4):
    raise NotImplementedError("write your pallas kernel here")


# speedup vs baseline: 6.1736x; 6.1736x over previous
import functools

import jax
import jax.numpy as jnp
from jax import lax
from jax.experimental import pallas as pl
from jax.experimental.pallas import tpu as pltpu
from jax.experimental.pallas import tpu_sc as plsc

N = 10000
E = 320000
D = 128
NPAD = 10112
K = 128
GC = 8
NG = 10
CPT = GC * NG
TPT = K * CPT
NW = 32
EPAD = TPT * NW
ROWS_PER_TILE = NPAD // 16
R = 2000
GRID = N // R


def _make_sc_agg(want_deg: bool):
    mesh = plsc.VectorSubcoreMesh(core_axis_name="c", subcore_axis_name="s")
    outs = [jax.ShapeDtypeStruct((2, NPAD, D), jnp.float32)]
    scratch = [
        pltpu.VMEM((K,), jnp.int32),
        pltpu.VMEM((K,), jnp.int32),
        pltpu.VMEM((K, D), jnp.float32),
        pltpu.VMEM_SHARED((NPAD, D), jnp.float32),
        pltpu.SemaphoreType.DMA,
    ]
    if want_deg:
        outs.append(jax.ShapeDtypeStruct((NW, NPAD), jnp.float32))
        scratch += [
            pltpu.VMEM((NPAD,), jnp.float32),
        ]

    @functools.partial(pl.kernel, out_type=outs, mesh=mesh,
                       compiler_params=pltpu.CompilerParams(
                           needs_layout_passes=False),
                       scratch_types=scratch)
    def sc_agg(*refs):
        if want_deg:
            (x_hbm, src_hbm, dst_hbm, z128_hbm,
             agg_out, deg_out, src_v, dst_v, rows_v, agg_sh, sem,
             deg_v) = refs
        else:
            (x_hbm, src_hbm, dst_hbm, z128_hbm,
             agg_out, src_v, dst_v, rows_v, agg_sh, sem) = refs
        c = lax.axis_index("c")
        s = lax.axis_index("s")
        w = c * 16 + s
        r0 = s * ROWS_PER_TILE
        pltpu.sync_copy(z128_hbm.at[pl.ds(r0, ROWS_PER_TILE)],
                        agg_sh.at[pl.ds(r0, ROWS_PER_TILE)])
        if want_deg:
            def zero(i, carry):
                deg_v[pl.ds(i * 16, 16)] = jnp.zeros((16,), jnp.float32)
                return carry

            lax.fori_loop(0, NPAD // 16, zero, 0)
        plsc.subcore_barrier()

        def body(j, carry):
            base = (w * CPT + j) * K
            pltpu.sync_copy(src_hbm.at[pl.ds(base, K)], src_v)
            pltpu.sync_copy(dst_hbm.at[pl.ds(base, K)], dst_v)
            pltpu.async_copy(x_hbm.at[src_v], rows_v, sem).wait()
            pltpu.sync_copy(rows_v, agg_sh.at[dst_v], add=True)
            if want_deg:
                def inner(i, carry2):
                    v = dst_v[pl.ds(i * 16, 16)]
                    cnt, last = plsc.scan_count(v)
                    plsc.addupdate_scatter(deg_v, [v],
                                           cnt.astype(jnp.float32),
                                           mask=last)
                    return carry2

                lax.fori_loop(0, K // 16, inner, carry)
            return carry

        lax.fori_loop(0, CPT, body, 0)
        plsc.subcore_barrier()
        pltpu.sync_copy(agg_sh.at[pl.ds(r0, ROWS_PER_TILE)],
                        agg_out.at[c, pl.ds(r0, ROWS_PER_TILE)])
        if want_deg:
            pltpu.sync_copy(deg_v, deg_out.at[w])

    return sc_agg


_sc_agg_deg = _make_sc_agg(True)
_sc_agg = _make_sc_agg(False)


def _layer1_body(x_ref, agg_ref, deg_ref, ws_ref, wn_ref, b_ref, h_ref):
    deg = jnp.maximum(jnp.sum(deg_ref[...], axis=1), 1.0)[:, None]
    mean = (agg_ref[0] + agg_ref[1]) / deg
    h = (jnp.dot(x_ref[...], ws_ref[...], preferred_element_type=jnp.float32)
         + jnp.dot(mean, wn_ref[...], preferred_element_type=jnp.float32)
         + b_ref[...])
    h_ref[...] = jnp.where(h >= 0, h, 0.01 * h)


def _tc_layer1(x, agg, deg, Ws, Wn, b):
    return pl.pallas_call(
        _layer1_body,
        grid=(GRID,),
        in_specs=[
            pl.BlockSpec((R, D), lambda i: (i, 0)),
            pl.BlockSpec((2, R, D), lambda i: (0, i, 0)),
            pl.BlockSpec((R, NW), lambda i: (i, 0)),
            pl.BlockSpec((D, D), lambda i: (0, 0)),
            pl.BlockSpec((D, D), lambda i: (0, 0)),
            pl.BlockSpec((1, D), lambda i: (0, 0)),
        ],
        out_specs=pl.BlockSpec((R, D), lambda i: (i, 0)),
        out_shape=jax.ShapeDtypeStruct((N, D), jnp.float32),
    )(x, agg, deg, Ws, Wn, b)


def _layer2_body(x_ref, agg_ref, deg_ref, ws_ref, wn_ref, b_ref,
                 mf_ref, w3a_ref, w3b_ref, b3_ref, w4_ref, b4_ref,
                 out_ref, acc_ref):
    i = pl.program_id(0)

    @pl.when(i == 0)
    def _():
        acc_ref[...] = jnp.zeros_like(acc_ref)

    deg = jnp.maximum(jnp.sum(deg_ref[...], axis=1), 1.0)[:, None]
    mean = (agg_ref[0] + agg_ref[1]) / deg
    h = (jnp.dot(x_ref[...], ws_ref[...], preferred_element_type=jnp.float32)
         + jnp.dot(mean, wn_ref[...], preferred_element_type=jnp.float32)
         + b_ref[...])
    h = jnp.where(h >= 0, h, 0.01 * h)
    acc_ref[...] += jnp.sum(h, axis=0, keepdims=True)

    @pl.when(i == GRID - 1)
    def _():
        g = acc_ref[...] / jnp.float32(N)
        r = (jnp.dot(g, w3a_ref[...], preferred_element_type=jnp.float32)
             + jnp.dot(mf_ref[...], w3b_ref[...],
                       preferred_element_type=jnp.float32)
             + b3_ref[...])
        r = jnp.maximum(r, 0.0)
        out_ref[...] = (jnp.dot(r, w4_ref[...],
                                preferred_element_type=jnp.float32)
                        + b4_ref[...])


def _tc_layer2(x, agg, deg, Ws, Wn, b, mf, W3a, W3b, b3, W4, b4):
    return pl.pallas_call(
        _layer2_body,
        grid=(GRID,),
        in_specs=[
            pl.BlockSpec((R, D), lambda i: (i, 0)),
            pl.BlockSpec((2, R, D), lambda i: (0, i, 0)),
            pl.BlockSpec((R, NW), lambda i: (i, 0)),
            pl.BlockSpec((D, D), lambda i: (0, 0)),
            pl.BlockSpec((D, D), lambda i: (0, 0)),
            pl.BlockSpec((1, D), lambda i: (0, 0)),
            pl.BlockSpec((1, 16), lambda i: (0, 0)),
            pl.BlockSpec((D, 64), lambda i: (0, 0)),
            pl.BlockSpec((16, 64), lambda i: (0, 0)),
            pl.BlockSpec((1, 64), lambda i: (0, 0)),
            pl.BlockSpec((64, 1), lambda i: (0, 0)),
            pl.BlockSpec((1, 1), lambda i: (0, 0)),
        ],
        out_specs=pl.BlockSpec((1, 1), lambda i: (0, 0)),
        out_shape=jax.ShapeDtypeStruct((1, 1), jnp.float32),
        scratch_shapes=[pltpu.VMEM((1, D), jnp.float32)],
    )(x, agg, deg, Ws, Wn, b, mf, W3a, W3b, b3, W4, b4)


def kernel(x, edge_index, manual_features, W1_self, W1_neigh, b1,
           W2_self, W2_neigh, b2, W3, b3, W4, b4):
    src = edge_index[0]
    dst = edge_index[1]
    pad_ar = jnp.arange(EPAD - E, dtype=jnp.int32)
    pad_s = pad_ar % N
    pad_d = N + pad_ar % (NPAD - N)
    src3 = jnp.concatenate([src, pad_s])
    dst3 = jnp.concatenate([dst, pad_d])
    z128 = jnp.zeros((NPAD, D), jnp.float32)

    agg1, deg = _sc_agg_deg(x, src3, dst3, z128)
    deg = deg.T
    h1 = _tc_layer1(x, agg1, deg, W1_self, W1_neigh, b1.reshape(1, D))
    (agg2,) = _sc_agg(h1, src3, dst3, z128)
    out = _tc_layer2(h1, agg2, deg, W2_self, W2_neigh, b2.reshape(1, D),
                     manual_features.reshape(1, 16),
                     W3[:D], W3[D:], b3.reshape(1, 64),
                     W4, b4.reshape(1, 1))
    return out.reshape(1)


# speedup vs baseline: 9.9283x; 1.6082x over previous
import functools

import jax
import jax.numpy as jnp
from jax import lax
from jax.experimental import pallas as pl
from jax.experimental.pallas import tpu as pltpu
from jax.experimental.pallas import tpu_sc as plsc

N = 10000
E = 320000
D = 128
NPAD = 10112
K = 128
G = 8
NG = 10
CPT = G * NG
TPT = K * CPT
NW = 32
EPAD = TPT * NW
ROWS_PER_TILE = NPAD // 16
R = 2000
GRID = N // R


def _make_sc_agg(want_deg: bool):
    mesh = plsc.VectorSubcoreMesh(core_axis_name="c", subcore_axis_name="s")
    outs = [jax.ShapeDtypeStruct((2, NPAD, D), jnp.float32)]
    scratch = [
        pltpu.VMEM((G, K), jnp.int32),
        pltpu.VMEM((G, K), jnp.int32),
        pltpu.VMEM((K, D), jnp.float32),
        pltpu.VMEM((K, D), jnp.float32),
        pltpu.VMEM_SHARED((NPAD, D), jnp.float32),
        pltpu.SemaphoreType.DMA,
        pltpu.SemaphoreType.DMA,
        pltpu.SemaphoreType.DMA,
        pltpu.SemaphoreType.DMA,
    ]
    if want_deg:
        outs.append(jax.ShapeDtypeStruct((NW, NPAD), jnp.float32))
        scratch += [
            pltpu.VMEM((NPAD,), jnp.float32),
        ]

    @functools.partial(pl.kernel, out_type=outs, mesh=mesh,
                       compiler_params=pltpu.CompilerParams(
                           needs_layout_passes=False),
                       scratch_types=scratch)
    def sc_agg(*refs):
        if want_deg:
            (x_hbm, src_hbm, dst_hbm, z128_hbm,
             agg_out, deg_out, src_g, dst_g, rows0, rows1, agg_sh,
             gsem0, gsem1, ssem0, ssem1, deg_v) = refs
        else:
            (x_hbm, src_hbm, dst_hbm, z128_hbm,
             agg_out, src_g, dst_g, rows0, rows1, agg_sh,
             gsem0, gsem1, ssem0, ssem1) = refs
        rows = (rows0, rows1)
        gsem = (gsem0, gsem1)
        ssem = (ssem0, ssem1)
        c = lax.axis_index("c")
        s = lax.axis_index("s")
        w = c * 16 + s
        r0 = s * ROWS_PER_TILE
        pltpu.sync_copy(z128_hbm.at[pl.ds(r0, ROWS_PER_TILE)],
                        agg_sh.at[pl.ds(r0, ROWS_PER_TILE)])
        if want_deg:
            def zero(i, carry):
                deg_v[pl.ds(i * 16, 16)] = jnp.zeros((16,), jnp.float32)
                return carry

            lax.fori_loop(0, NPAD // 16, zero, 0)
        plsc.subcore_barrier()

        def group(g, carry):
            gi = w * NG + g
            pltpu.sync_copy(src_hbm.at[gi], src_g)
            pltpu.sync_copy(dst_hbm.at[gi], dst_g)
            shandles = [None, None]
            for jj in range(G):
                b = jj % 2
                if shandles[b] is not None:
                    shandles[b].wait()
                gh = pltpu.async_copy(x_hbm.at[src_g.at[jj]], rows[b],
                                      gsem[b])
                if want_deg:
                    def inner(i, carry2, jj=jj):
                        v = dst_g[jj, pl.ds(i * 16, 16)]
                        cnt, last = plsc.scan_count(v)
                        plsc.addupdate_scatter(deg_v, [v],
                                               cnt.astype(jnp.float32),
                                               mask=last)
                        return carry2

                    lax.fori_loop(0, K // 16, inner, 0)
                gh.wait()
                shandles[b] = pltpu.async_copy(rows[b],
                                               agg_sh.at[dst_g.at[jj]],
                                               ssem[b], add=True)
            shandles[0].wait()
            shandles[1].wait()
            return carry

        lax.fori_loop(0, NG, group, 0)
        plsc.subcore_barrier()
        pltpu.sync_copy(agg_sh.at[pl.ds(r0, ROWS_PER_TILE)],
                        agg_out.at[c, pl.ds(r0, ROWS_PER_TILE)])
        if want_deg:
            pltpu.sync_copy(deg_v, deg_out.at[w])

    return sc_agg


_sc_agg_deg = _make_sc_agg(True)
_sc_agg = _make_sc_agg(False)


def _layer1_body(x_ref, agg_ref, deg_ref, ws_ref, wn_ref, b_ref, h_ref):
    deg = jnp.maximum(jnp.sum(deg_ref[...], axis=1), 1.0)[:, None]
    mean = (agg_ref[0] + agg_ref[1]) / deg
    h = (jnp.dot(x_ref[...], ws_ref[...], preferred_element_type=jnp.float32)
         + jnp.dot(mean, wn_ref[...], preferred_element_type=jnp.float32)
         + b_ref[...])
    h_ref[...] = jnp.where(h >= 0, h, 0.01 * h)


def _tc_layer1(x, agg, deg, Ws, Wn, b):
    return pl.pallas_call(
        _layer1_body,
        grid=(GRID,),
        in_specs=[
            pl.BlockSpec((R, D), lambda i: (i, 0)),
            pl.BlockSpec((2, R, D), lambda i: (0, i, 0)),
            pl.BlockSpec((R, NW), lambda i: (i, 0)),
            pl.BlockSpec((D, D), lambda i: (0, 0)),
            pl.BlockSpec((D, D), lambda i: (0, 0)),
            pl.BlockSpec((1, D), lambda i: (0, 0)),
        ],
        out_specs=pl.BlockSpec((R, D), lambda i: (i, 0)),
        out_shape=jax.ShapeDtypeStruct((N, D), jnp.float32),
    )(x, agg, deg, Ws, Wn, b)


def _layer2_body(x_ref, agg_ref, deg_ref, ws_ref, wn_ref, b_ref,
                 mf_ref, w3a_ref, w3b_ref, b3_ref, w4_ref, b4_ref,
                 out_ref, acc_ref):
    i = pl.program_id(0)

    @pl.when(i == 0)
    def _():
        acc_ref[...] = jnp.zeros_like(acc_ref)

    deg = jnp.maximum(jnp.sum(deg_ref[...], axis=1), 1.0)[:, None]
    mean = (agg_ref[0] + agg_ref[1]) / deg
    h = (jnp.dot(x_ref[...], ws_ref[...], preferred_element_type=jnp.float32)
         + jnp.dot(mean, wn_ref[...], preferred_element_type=jnp.float32)
         + b_ref[...])
    h = jnp.where(h >= 0, h, 0.01 * h)
    acc_ref[...] += jnp.sum(h, axis=0, keepdims=True)

    @pl.when(i == GRID - 1)
    def _():
        g = acc_ref[...] / jnp.float32(N)
        r = (jnp.dot(g, w3a_ref[...], preferred_element_type=jnp.float32)
             + jnp.dot(mf_ref[...], w3b_ref[...],
                       preferred_element_type=jnp.float32)
             + b3_ref[...])
        r = jnp.maximum(r, 0.0)
        out_ref[...] = (jnp.dot(r, w4_ref[...],
                                preferred_element_type=jnp.float32)
                        + b4_ref[...])


def _tc_layer2(x, agg, deg, Ws, Wn, b, mf, W3a, W3b, b3, W4, b4):
    return pl.pallas_call(
        _layer2_body,
        grid=(GRID,),
        in_specs=[
            pl.BlockSpec((R, D), lambda i: (i, 0)),
            pl.BlockSpec((2, R, D), lambda i: (0, i, 0)),
            pl.BlockSpec((R, NW), lambda i: (i, 0)),
            pl.BlockSpec((D, D), lambda i: (0, 0)),
            pl.BlockSpec((D, D), lambda i: (0, 0)),
            pl.BlockSpec((1, D), lambda i: (0, 0)),
            pl.BlockSpec((1, 16), lambda i: (0, 0)),
            pl.BlockSpec((D, 64), lambda i: (0, 0)),
            pl.BlockSpec((16, 64), lambda i: (0, 0)),
            pl.BlockSpec((1, 64), lambda i: (0, 0)),
            pl.BlockSpec((64, 1), lambda i: (0, 0)),
            pl.BlockSpec((1, 1), lambda i: (0, 0)),
        ],
        out_specs=pl.BlockSpec((1, 1), lambda i: (0, 0)),
        out_shape=jax.ShapeDtypeStruct((1, 1), jnp.float32),
        scratch_shapes=[pltpu.VMEM((1, D), jnp.float32)],
    )(x, agg, deg, Ws, Wn, b, mf, W3a, W3b, b3, W4, b4)


def kernel(x, edge_index, manual_features, W1_self, W1_neigh, b1,
           W2_self, W2_neigh, b2, W3, b3, W4, b4):
    src = edge_index[0]
    dst = edge_index[1]
    pad_ar = jnp.arange(EPAD - E, dtype=jnp.int32)
    pad_s = pad_ar % N
    pad_d = N + pad_ar % (NPAD - N)
    src3 = jnp.concatenate([src, pad_s]).reshape(NW * NG, G, K)
    dst3 = jnp.concatenate([dst, pad_d]).reshape(NW * NG, G, K)
    z128 = jnp.zeros((NPAD, D), jnp.float32)

    agg1, deg = _sc_agg_deg(x, src3, dst3, z128)
    deg = deg.T
    h1 = _tc_layer1(x, agg1, deg, W1_self, W1_neigh, b1.reshape(1, D))
    (agg2,) = _sc_agg(h1, src3, dst3, z128)
    out = _tc_layer2(h1, agg2, deg, W2_self, W2_neigh, b2.reshape(1, D),
                     manual_features.reshape(1, 16),
                     W3[:D], W3[D:], b3.reshape(1, 64),
                     W4, b4.reshape(1, 1))
    return out.reshape(1)


# speedup vs baseline: 10.7745x; 1.0852x over previous
import functools

import jax
import jax.numpy as jnp
from jax import lax
from jax.experimental import pallas as pl
from jax.experimental.pallas import tpu as pltpu
from jax.experimental.pallas import tpu_sc as plsc

N = 10000
E = 320000
D = 128
NPAD = 10112
K = 128
G = 8
NG = 10
CPT = G * NG
TPT = K * CPT
NW = 32
EPAD = TPT * NW
ROWS_PER_TILE = NPAD // 16
R = 2000
GRID = N // R


def _make_sc_agg(want_deg: bool):
    mesh = plsc.VectorSubcoreMesh(core_axis_name="c", subcore_axis_name="s")
    outs = [jax.ShapeDtypeStruct((2, NPAD, D), jnp.float32)]
    scratch = [
        pltpu.VMEM((G, K), jnp.int32),
        pltpu.VMEM((G, K), jnp.int32),
        pltpu.VMEM((K, D), jnp.float32),
        pltpu.VMEM((K, D), jnp.float32),
        pltpu.VMEM_SHARED((NPAD, D), jnp.float32),
        pltpu.SemaphoreType.DMA,
        pltpu.SemaphoreType.DMA,
        pltpu.SemaphoreType.DMA,
        pltpu.SemaphoreType.DMA,
    ]
    if want_deg:
        outs.append(jax.ShapeDtypeStruct((NW, NPAD), jnp.float32))
        scratch += [
            pltpu.VMEM((NPAD,), jnp.float32),
        ]

    @functools.partial(pl.kernel, out_type=outs, mesh=mesh,
                       compiler_params=pltpu.CompilerParams(
                           needs_layout_passes=False),
                       scratch_types=scratch)
    def sc_agg(*refs):
        if want_deg:
            (x_hbm, src_hbm, dst_hbm, z128_hbm,
             agg_out, deg_out, src_g, dst_g, rows0, rows1, agg_sh,
             gsem0, gsem1, ssem0, ssem1, deg_v) = refs
        else:
            (x_hbm, src_hbm, dst_hbm, z128_hbm,
             agg_out, src_g, dst_g, rows0, rows1, agg_sh,
             gsem0, gsem1, ssem0, ssem1) = refs
        rows = (rows0, rows1)
        gsem = (gsem0, gsem1)
        ssem = (ssem0, ssem1)
        c = lax.axis_index("c")
        s = lax.axis_index("s")
        w = c * 16 + s
        r0 = s * ROWS_PER_TILE
        pltpu.sync_copy(z128_hbm.at[pl.ds(r0, ROWS_PER_TILE)],
                        agg_sh.at[pl.ds(r0, ROWS_PER_TILE)])
        if want_deg:
            def zero(i, carry):
                deg_v[pl.ds(i * 16, 16)] = jnp.zeros((16,), jnp.float32)
                return carry

            lax.fori_loop(0, NPAD // 16, zero, 0)
        plsc.subcore_barrier()

        def group(g, carry):
            gi = w * NG + g
            pltpu.sync_copy(src_hbm.at[gi], src_g)
            pltpu.sync_copy(dst_hbm.at[gi], dst_g)
            handles = [None, None]
            handles[0] = pltpu.async_copy(x_hbm.at[src_g.at[0]], rows[0],
                                          gsem[0])
            for jj in range(G):
                b = jj % 2
                handles[b].wait()
                if jj + 1 < G:
                    nb = (jj + 1) % 2
                    handles[nb] = pltpu.async_copy(
                        x_hbm.at[src_g.at[jj + 1]], rows[nb], gsem[nb])
                if want_deg:
                    def inner(i, carry2, jj=jj):
                        v = dst_g[jj, pl.ds(i * 16, 16)]
                        cnt, last = plsc.scan_count(v)
                        plsc.addupdate_scatter(deg_v, [v],
                                               cnt.astype(jnp.float32),
                                               mask=last)
                        return carry2

                    lax.fori_loop(0, K // 16, inner, 0)
                pltpu.async_copy(rows[b], agg_sh.at[dst_g.at[jj]],
                                 ssem[b], add=True).wait()
            return carry

        lax.fori_loop(0, NG, group, 0)
        plsc.subcore_barrier()
        pltpu.sync_copy(agg_sh.at[pl.ds(r0, ROWS_PER_TILE)],
                        agg_out.at[c, pl.ds(r0, ROWS_PER_TILE)])
        if want_deg:
            pltpu.sync_copy(deg_v, deg_out.at[w])

    return sc_agg


_sc_agg_deg = _make_sc_agg(True)
_sc_agg = _make_sc_agg(False)


def _layer1_body(x_ref, agg_ref, deg_ref, ws_ref, wn_ref, b_ref, h_ref):
    deg = jnp.maximum(jnp.sum(deg_ref[...], axis=1), 1.0)[:, None]
    mean = (agg_ref[0] + agg_ref[1]) / deg
    h = (jnp.dot(x_ref[...], ws_ref[...],
                 preferred_element_type=jnp.float32,
                 precision=lax.Precision.HIGHEST)
         + jnp.dot(mean, wn_ref[...],
                   preferred_element_type=jnp.float32,
                   precision=lax.Precision.HIGHEST)
         + b_ref[...])
    h_ref[...] = jnp.where(h >= 0, h, 0.01 * h)


def _tc_layer1(x, agg, deg, Ws, Wn, b):
    return pl.pallas_call(
        _layer1_body,
        grid=(GRID,),
        in_specs=[
            pl.BlockSpec((R, D), lambda i: (i, 0)),
            pl.BlockSpec((2, R, D), lambda i: (0, i, 0)),
            pl.BlockSpec((R, NW), lambda i: (i, 0)),
            pl.BlockSpec((D, D), lambda i: (0, 0)),
            pl.BlockSpec((D, D), lambda i: (0, 0)),
            pl.BlockSpec((1, D), lambda i: (0, 0)),
        ],
        out_specs=pl.BlockSpec((R, D), lambda i: (i, 0)),
        out_shape=jax.ShapeDtypeStruct((N, D), jnp.float32),
    )(x, agg, deg, Ws, Wn, b)


def _layer2_body(x_ref, agg_ref, deg_ref, ws_ref, wn_ref, b_ref,
                 mf_ref, w3a_ref, w3b_ref, b3_ref, w4_ref, b4_ref,
                 out_ref, acc_ref):
    i = pl.program_id(0)

    @pl.when(i == 0)
    def _():
        acc_ref[...] = jnp.zeros_like(acc_ref)

    deg = jnp.maximum(jnp.sum(deg_ref[...], axis=1), 1.0)[:, None]
    mean = (agg_ref[0] + agg_ref[1]) / deg
    h = (jnp.dot(x_ref[...], ws_ref[...],
                 preferred_element_type=jnp.float32,
                 precision=lax.Precision.HIGHEST)
         + jnp.dot(mean, wn_ref[...],
                   preferred_element_type=jnp.float32,
                   precision=lax.Precision.HIGHEST)
         + b_ref[...])
    h = jnp.where(h >= 0, h, 0.01 * h)
    acc_ref[...] += jnp.sum(h, axis=0, keepdims=True)

    @pl.when(i == GRID - 1)
    def _():
        g = acc_ref[...] / jnp.float32(N)
        r = (jnp.dot(g, w3a_ref[...],
                     preferred_element_type=jnp.float32,
                     precision=lax.Precision.HIGHEST)
             + jnp.dot(mf_ref[...], w3b_ref[...],
                       preferred_element_type=jnp.float32,
                       precision=lax.Precision.HIGHEST)
             + b3_ref[...])
        r = jnp.maximum(r, 0.0)
        out_ref[...] = (jnp.dot(r, w4_ref[...],
                                preferred_element_type=jnp.float32,
                                precision=lax.Precision.HIGHEST)
                        + b4_ref[...])


def _tc_layer2(x, agg, deg, Ws, Wn, b, mf, W3a, W3b, b3, W4, b4):
    return pl.pallas_call(
        _layer2_body,
        grid=(GRID,),
        in_specs=[
            pl.BlockSpec((R, D), lambda i: (i, 0)),
            pl.BlockSpec((2, R, D), lambda i: (0, i, 0)),
            pl.BlockSpec((R, NW), lambda i: (i, 0)),
            pl.BlockSpec((D, D), lambda i: (0, 0)),
            pl.BlockSpec((D, D), lambda i: (0, 0)),
            pl.BlockSpec((1, D), lambda i: (0, 0)),
            pl.BlockSpec((1, 16), lambda i: (0, 0)),
            pl.BlockSpec((D, 64), lambda i: (0, 0)),
            pl.BlockSpec((16, 64), lambda i: (0, 0)),
            pl.BlockSpec((1, 64), lambda i: (0, 0)),
            pl.BlockSpec((64, 1), lambda i: (0, 0)),
            pl.BlockSpec((1, 1), lambda i: (0, 0)),
        ],
        out_specs=pl.BlockSpec((1, 1), lambda i: (0, 0)),
        out_shape=jax.ShapeDtypeStruct((1, 1), jnp.float32),
        scratch_shapes=[pltpu.VMEM((1, D), jnp.float32)],
    )(x, agg, deg, Ws, Wn, b, mf, W3a, W3b, b3, W4, b4)


def kernel(x, edge_index, manual_features, W1_self, W1_neigh, b1,
           W2_self, W2_neigh, b2, W3, b3, W4, b4):
    src = edge_index[0]
    dst = edge_index[1]
    pad_ar = jnp.arange(EPAD - E, dtype=jnp.int32)
    pad_s = pad_ar % N
    pad_d = N + pad_ar % (NPAD - N)
    src3 = jnp.concatenate([src, pad_s]).reshape(NW * NG, G, K)
    dst3 = jnp.concatenate([dst, pad_d]).reshape(NW * NG, G, K)
    z128 = jnp.zeros((NPAD, D), jnp.float32)

    agg1, deg = _sc_agg_deg(x, src3, dst3, z128)
    deg = deg.T
    h1 = _tc_layer1(x, agg1, deg, W1_self, W1_neigh, b1.reshape(1, D))
    (agg2,) = _sc_agg(h1, src3, dst3, z128)
    out = _tc_layer2(h1, agg2, deg, W2_self, W2_neigh, b2.reshape(1, D),
                     manual_features.reshape(1, 16),
                     W3[:D], W3[D:], b3.reshape(1, 64),
                     W4, b4.reshape(1, 1))
    return out.reshape(1)
